# baseline (speedup 1.0000x reference)
import functools

import jax
import jax.numpy as jnp
from jax import lax
from jax.experimental import pallas as pl
from jax.experimental.pallas import tpu as pltpu
from jax.experimental.pallas import tpu_sc as plsc

N = 10000
E = 320000
D = 128
NW = 32
NP = 10240
PT = NP // NW
EC = 128
RING = 32
NEG = -3.4e38


def _mesh():
    return plsc.VectorSubcoreMesh(core_axis_name="c", subcore_axis_name="s")


def _drain(ls_v, lo_v, le_v, idx_v, rowbuf_v, t1_v, t2_v, sem, x_hbm, acc_v,
           rpos, nvalid, weighted):
    iota = lax.iota(jnp.int32, 16)
    ridx = (jnp.full((16,), rpos, jnp.int32) + iota) & (RING - 1)
    valid = iota < jnp.full((16,), nvalid, jnp.int32)
    sg = plsc.load_gather(ls_v, [ridx])
    og = plsc.load_gather(lo_v, [ridx])
    sg = jnp.where(valid, sg, 0)
    og = jnp.where(valid, og, 0)
    if weighted:
        eg = plsc.load_gather(le_v, [ridx])
    idx_v[...] = sg
    pltpu.async_copy(x_hbm.at[idx_v], rowbuf_v, sem).wait()
    okey = jnp.where(valid, og, PT + iota)
    so, perm = plsc.sort_key_val(okey, iota)
    t1_v[...] = so
    prev = plsc.load_gather(t1_v, [jnp.maximum(iota - 1, 0)])
    fo = jnp.logical_or(iota == 0, so != prev)
    bse = plsc.cummax(jnp.where(fo, iota, 0))
    plsc.store_scatter(t2_v, [perm], iota - bse)
    rk = jnp.where(valid, t2_v[...], -1)
    nr = jnp.max(rk) + 1

    def round_body(r, _):
        msk = rk == jnp.full((16,), r, jnp.int32)

        def col_step(cs, _2):
            for j in range(16):
                jj = jnp.full((16,), cs * 16 + j, jnp.int32)
                hv = plsc.load_gather(rowbuf_v, [iota, jj], mask=msk)
                val = eg * hv if weighted else hv
                cur = plsc.load_gather(acc_v, [og, jj], mask=msk)
                plsc.store_scatter(acc_v, [og, jj], jnp.maximum(cur, val),
                                   mask=msk)
            return 0

        lax.fori_loop(0, D // 16, col_step, 0)
        return 0

    lax.fori_loop(0, nr, round_body, 0)


def _gat_scan_body(src_hbm, dst_hbm, asrc_hbm, adst_hbm, h_hbm, mv_hbm,
                   ninit_hbm, zrep0_hbm, maxout_hbm, z_hbm,
                   asrc_v, adst_v, acc_v, zrep_v, srcb_v, dstb_v,
                   ls_v, lo_v, le_v, idx_v, rowbuf_v, t1_v, t2_v, zout_v,
                   mvf_v, sem):
    cid = lax.axis_index("c")
    sid = lax.axis_index("s")
    wid = sid * 2 + cid
    base = wid * PT
    iota = lax.iota(jnp.int32, 16)
    pltpu.sync_copy(asrc_hbm, asrc_v)
    pltpu.sync_copy(adst_hbm, adst_v)
    pltpu.sync_copy(ninit_hbm, acc_v)
    pltpu.sync_copy(mv_hbm, mvf_v)
    pltpu.sync_copy(zrep0_hbm, zrep_v)
    mv = mvf_v[...]

    def drain_full(rp):
        _drain(ls_v, lo_v, le_v, idx_v, rowbuf_v, t1_v, t2_v, sem, h_hbm,
               acc_v, rp, jnp.int32(16), True)
        return rp + 16

    def chunk(ci, carry):
        wpos, rpos = carry
        pltpu.sync_copy(src_hbm.at[pl.ds(ci * EC, EC)], srcb_v)
        pltpu.sync_copy(dst_hbm.at[pl.ds(ci * EC, EC)], dstb_v)
        for g in range(EC // 16):
            s = srcb_v[pl.ds(g * 16, 16)]
            d = dstb_v[pl.ds(g * 16, 16)]
            bb = jnp.full((16,), base, jnp.int32)
            own = jnp.logical_and(d >= bb, d < bb + PT)
            av = plsc.load_gather(asrc_v, [s])
            bv = plsc.load_gather(adst_v, [d])
            q = av + bv
            e = jnp.maximum(q, 0.0) + 0.2 * jnp.minimum(q, 0.0)
            expe = jnp.exp(e - mv)
            off = jnp.where(own, d - bb, 0)
            curz = plsc.load_gather(zrep_v, [iota, off], mask=own)
            plsc.store_scatter(zrep_v, [iota, off], curz + expe, mask=own)
            owni = own.astype(jnp.int32)
            pc = plsc.cumsum(owni)
            pos = (jnp.full((16,), wpos, jnp.int32) + pc - 1) & (RING - 1)
            plsc.store_scatter(ls_v, [pos], s, mask=own)
            plsc.store_scatter(lo_v, [pos], off, mask=own)
            plsc.store_scatter(le_v, [pos], expe, mask=own)
            wpos = wpos + jnp.max(pc)
            rpos = lax.cond(wpos - rpos >= 16, drain_full, lambda rp: rp, rpos)
        return wpos, rpos

    wpos, rpos = lax.fori_loop(0, E // EC, chunk,
                               (jnp.int32(0), jnp.int32(0)))

    def drain_tail(rp):
        _drain(ls_v, lo_v, le_v, idx_v, rowbuf_v, t1_v, t2_v, sem, h_hbm,
               acc_v, rp, wpos - rp, True)
        return rp

    rpos = lax.cond(wpos > rpos, drain_tail, lambda rp: rp, rpos)

    def zfin(g, _):
        zacc = zrep_v[0, pl.ds(g * 16, 16)]
        for r in range(1, 16):
            zacc = zacc + zrep_v[r, pl.ds(g * 16, 16)]
        zout_v[pl.ds(g * 16, 16)] = zacc
        return 0

    lax.fori_loop(0, PT // 16, zfin, 0)
    pltpu.sync_copy(zout_v, z_hbm.at[pl.ds(base, PT)])
    pltpu.sync_copy(acc_v, maxout_hbm.at[pl.ds(base, PT)])


def _gat_sc(src, dst, a_src, a_dst, h, mval):
    asrc_p = jnp.zeros((NP,), jnp.float32).at[:N].set(a_src)
    adst_p = jnp.zeros((NP,), jnp.float32).at[:N].set(a_dst)
    mvec = jnp.full((16,), mval, jnp.float32)
    ninit = jnp.full((PT, D), NEG, jnp.float32)
    zrep0 = jnp.zeros((16, PT), jnp.float32)
    k = pl.kernel(
        _gat_scan_body,
        out_type=(jax.ShapeDtypeStruct((NP, D), jnp.float32),
                  jax.ShapeDtypeStruct((NP,), jnp.float32)),
        mesh=_mesh(),
        compiler_params=pltpu.CompilerParams(needs_layout_passes=False),
        scratch_types=[
            pltpu.VMEM((NP,), jnp.float32),
            pltpu.VMEM((NP,), jnp.float32),
            pltpu.VMEM((PT, D), jnp.float32),
            pltpu.VMEM((16, PT), jnp.float32),
            pltpu.VMEM((EC,), jnp.int32),
            pltpu.VMEM((EC,), jnp.int32),
            pltpu.VMEM((RING,), jnp.int32),
            pltpu.VMEM((RING,), jnp.int32),
            pltpu.VMEM((RING,), jnp.float32),
            pltpu.VMEM((16,), jnp.int32),
            pltpu.VMEM((16, D), jnp.float32),
            pltpu.VMEM((16,), jnp.int32),
            pltpu.VMEM((16,), jnp.int32),
            pltpu.VMEM((PT,), jnp.float32),
            pltpu.VMEM((16,), jnp.float32),
            pltpu.SemaphoreType.DMA,
        ],
    )
    return k(src, dst, asrc_p, adst_p, h, mvec, ninit, zrep0)


def _segmax_scan_body(src_hbm, dst_hbm, x_hbm, ninit_hbm, out_hbm,
                      acc_v, srcb_v, dstb_v, ls_v, lo_v, idx_v, rowbuf_v,
                      t1_v, t2_v, sem):
    cid = lax.axis_index("c")
    sid = lax.axis_index("s")
    wid = sid * 2 + cid
    base = wid * PT
    pltpu.sync_copy(ninit_hbm, acc_v)

    def drain_full(rp):
        _drain(ls_v, lo_v, None, idx_v, rowbuf_v, t1_v, t2_v, sem, x_hbm,
               acc_v, rp, jnp.int32(16), False)
        return rp + 16

    def chunk(ci, carry):
        wpos, rpos = carry
        pltpu.sync_copy(src_hbm.at[pl.ds(ci * EC, EC)], srcb_v)
        pltpu.sync_copy(dst_hbm.at[pl.ds(ci * EC, EC)], dstb_v)
        for g in range(EC // 16):
            s = srcb_v[pl.ds(g * 16, 16)]
            d = dstb_v[pl.ds(g * 16, 16)]
            bb = jnp.full((16,), base, jnp.int32)
            own = jnp.logical_and(d >= bb, d < bb + PT)
            off = jnp.where(own, d - bb, 0)
            owni = own.astype(jnp.int32)
            pc = plsc.cumsum(owni)
            pos = (jnp.full((16,), wpos, jnp.int32) + pc - 1) & (RING - 1)
            plsc.store_scatter(ls_v, [pos], s, mask=own)
            plsc.store_scatter(lo_v, [pos], off, mask=own)
            wpos = wpos + jnp.max(pc)
            rpos = lax.cond(wpos - rpos >= 16, drain_full, lambda rp: rp, rpos)
        return wpos, rpos

    wpos, rpos = lax.fori_loop(0, E // EC, chunk,
                               (jnp.int32(0), jnp.int32(0)))

    def drain_tail(rp):
        _drain(ls_v, lo_v, None, idx_v, rowbuf_v, t1_v, t2_v, sem, x_hbm,
               acc_v, rp, wpos - rp, False)
        return rp

    rpos = lax.cond(wpos > rpos, drain_tail, lambda rp: rp, rpos)
    pltpu.sync_copy(acc_v, out_hbm.at[pl.ds(base, PT)])


def _segmax_sc(src, dst, x):
    ninit = jnp.full((PT, D), NEG, jnp.float32)
    k = pl.kernel(
        _segmax_scan_body,
        out_type=jax.ShapeDtypeStruct((NP, D), jnp.float32),
        mesh=_mesh(),
        compiler_params=pltpu.CompilerParams(needs_layout_passes=False),
        scratch_types=[
            pltpu.VMEM((PT, D), jnp.float32),
            pltpu.VMEM((EC,), jnp.int32),
            pltpu.VMEM((EC,), jnp.int32),
            pltpu.VMEM((RING,), jnp.int32),
            pltpu.VMEM((RING,), jnp.int32),
            pltpu.VMEM((16,), jnp.int32),
            pltpu.VMEM((16, D), jnp.float32),
            pltpu.VMEM((16,), jnp.int32),
            pltpu.VMEM((16,), jnp.int32),
            pltpu.SemaphoreType.DMA,
        ],
    )
    return k(src, dst, x, ninit)


def _segsum_body(src_hbm, dst_hbm, x_hbm, zrows_hbm, out_hbm,
                 srcb_v, dstb_v, rows_v, sem, shared_v):
    cid = lax.axis_index("c")
    sid = lax.axis_index("s")
    wid = sid * 2 + cid
    sl = NP // 16
    pltpu.sync_copy(zrows_hbm, shared_v.at[pl.ds(sid * sl, sl)])
    plsc.subcore_barrier()
    nch = jnp.where(wid < (E // EC) % NW, (E // EC) // NW + 1, (E // EC) // NW)

    def body(i, _):
        ci = wid + i * NW
        pltpu.sync_copy(src_hbm.at[pl.ds(ci * EC, EC)], srcb_v)
        pltpu.sync_copy(dst_hbm.at[pl.ds(ci * EC, EC)], dstb_v)
        pltpu.async_copy(x_hbm.at[srcb_v], rows_v, sem).wait()
        pltpu.sync_copy(rows_v, shared_v.at[dstb_v], add=True)
        return 0

    lax.fori_loop(0, nch, body, 0)
    plsc.subcore_barrier()
    pltpu.sync_copy(shared_v.at[pl.ds(sid * sl, sl)],
                    out_hbm.at[cid, pl.ds(sid * sl, sl)])


def _segsum_sc(src, dst, x):
    zrows = jnp.zeros((NP // 16, D), jnp.float32)
    k = pl.kernel(
        _segsum_body,
        out_type=jax.ShapeDtypeStruct((2, NP, D), jnp.float32),
        mesh=_mesh(),
        compiler_params=pltpu.CompilerParams(needs_layout_passes=False),
        scratch_types=[
            pltpu.VMEM((EC,), jnp.int32),
            pltpu.VMEM((EC,), jnp.int32),
            pltpu.VMEM((EC, D), jnp.float32),
            pltpu.SemaphoreType.DMA,
            pltpu.VMEM_SHARED((NP, D), jnp.float32),
        ],
    )
    parts = k(src, dst, x, zrows)
    return parts[0] + parts[1]


def _bn(x, g, b, eps=1e-5):
    mu = x.mean(axis=0)
    var = x.var(axis=0)
    return (x - mu) / jnp.sqrt(var + eps) * g + b


def _prelu(x, a):
    return jnp.maximum(x, 0.0) + a * jnp.minimum(x, 0.0)


def _lrelu(x):
    return jnp.maximum(x, 0.0) + 0.2 * jnp.minimum(x, 0.0)


_USE_GAT_SC = True
_USE_SEGMAX_SC = True


def _tower(x, edge_index, p, pr):
    src, dst = edge_index[0], edge_index[1]
    h = x @ p[pr + 'W_gat']
    a_src = (h * p[pr + 'att_src']).sum(axis=-1)
    a_dst = (h * p[pr + 'att_dst']).sum(axis=-1)
    mval = _lrelu(a_src.max() + a_dst.max())
    expe_self = jnp.exp(_lrelu(a_src + a_dst) - mval)
    if _USE_GAT_SC:
        maxout, z = _gat_sc(src, dst, a_src, a_dst, h, mval)
        maxout, z = maxout[:N], z[:N]
    else:
        e = _lrelu(a_src[src] + a_dst[dst])
        expe = jnp.exp(e - mval)
        z = jax.ops.segment_sum(expe, dst, num_segments=N)
        maxout = jax.ops.segment_max(expe[:, None] * h[src], dst,
                                     num_segments=N)
    denom = z + expe_self + 1e-16 * jnp.exp(-mval)
    out = jnp.maximum(maxout, expe_self[:, None] * h) / denom[:, None]
    out = out + p[pr + 'b_gat']
    out = _prelu(_bn(out, p[pr + 'bn1_g'], p[pr + 'bn1_b']), p[pr + 'prelu1'])
    if _USE_SEGMAX_SC:
        nbr = _segmax_sc(src, dst, out)[:N]
    else:
        nbr = jax.ops.segment_max(out[src], dst, num_segments=N)
        nbr = jnp.where(jnp.isneginf(nbr), NEG, nbr)
    nbr = jnp.where(nbr < -1e37, 0.0, nbr)
    h2 = nbr @ p[pr + 'W_sage_l'] + p[pr + 'b_sage'] + out @ p[pr + 'W_sage_r']
    h2 = _prelu(_bn(h2, p[pr + 'bn2_g'], p[pr + 'bn2_b']), p[pr + 'prelu2'])
    agg = _segsum_sc(src, dst, h2)[:N]
    score = (agg @ p[pr + 'Wp_rel'] + p[pr + 'bp_rel'] +
             h2 @ p[pr + 'Wp_root']).reshape(-1)
    k = (N + 1) // 2
    _, perm = jax.lax.top_k(score, k)
    return jnp.max(h2[perm] * jnp.tanh(score[perm])[:, None], axis=0)


def _head_body(ae_ref, ge_ref, addf_ref, fc1w_ref, fc1b_ref, pr3_ref,
               fc2w_ref, fc2b_ref, out_ref):
    pool = jnp.concatenate([ae_ref[...], ge_ref[...], addf_ref[...]])[None, :]
    hid = pool @ fc1w_ref[...] + fc1b_ref[...][None, :]
    a3 = pr3_ref[0]
    hid = jnp.maximum(hid, 0.0) + a3 * jnp.minimum(hid, 0.0)
    out = jnp.exp(hid @ fc2w_ref[...] + fc2b_ref[...][None, :])
    out_ref[...] = out[0]


def kernel(A_x, A_edge_index, A_batch, A_W_gat, A_att_src, A_att_dst, A_b_gat, A_bn1_g, A_bn1_b, A_prelu1, A_W_sage_l, A_W_sage_r, A_b_sage, A_bn2_g, A_bn2_b, A_prelu2, A_Wp_rel, A_bp_rel, A_Wp_root, G_x, G_edge_index, G_batch, G_W_gat, G_att_src, G_att_dst, G_b_gat, G_bn1_g, G_bn1_b, G_prelu1, G_W_sage_l, G_W_sage_r, G_b_sage, G_bn2_g, G_bn2_b, G_prelu2, G_Wp_rel, G_bp_rel, G_Wp_root, add_features, fc1_W, fc1_b, prelu3, fc2_W, fc2_b):
    kw = dict(locals())
    pA = {k: v for k, v in kw.items() if k.startswith('A_')}
    pG = {k: v for k, v in kw.items() if k.startswith('G_')}
    ae = _tower(A_x, A_edge_index, pA, 'A_')
    ge = _tower(G_x, G_edge_index, pG, 'G_')
    out = pl.pallas_call(
        _head_body,
        out_shape=jax.ShapeDtypeStruct((1,), jnp.float32),
    )(ae, ge, add_features, fc1_W, fc1_b, prelu3, fc2_W, fc2_b)
    return out


# speedup vs baseline: 1.3365x; 1.3365x over previous
import jax
import jax.numpy as jnp
from jax import lax
from jax.experimental import pallas as pl
from jax.experimental.pallas import tpu as pltpu
from jax.experimental.pallas import tpu_sc as plsc

N = 10000
E = 320000
D = 128
NT = 16
NP = 10240
PT = NP // NT
EC = 128
RING = 32
NEG = -3.4e38


def _mesh():
    return plsc.VectorSubcoreMesh(core_axis_name="c", subcore_axis_name="s")


def _params():
    return pltpu.CompilerParams(needs_layout_passes=False)


def _drain(ls_v, lo_v, le_v, idx_v, rowbuf_v, t1_v, t2_v, sem, x_hbm, acc_v,
           rowoff, rpos, nvalid, weighted):
    iota = lax.iota(jnp.int32, 16)
    ridx = (jnp.full((16,), rpos, jnp.int32) + iota) & (RING - 1)
    valid = iota < jnp.full((16,), nvalid, jnp.int32)
    sg = plsc.load_gather(ls_v, [ridx])
    og = plsc.load_gather(lo_v, [ridx])
    sg = jnp.where(valid, sg, 0)
    og = jnp.where(valid, og, 0)
    if weighted:
        eg = plsc.load_gather(le_v, [ridx])
    idx_v[...] = sg + jnp.full((16,), rowoff, jnp.int32)
    pltpu.async_copy(x_hbm.at[idx_v], rowbuf_v, sem).wait()
    okey = jnp.where(valid, og, PT + iota)
    so, perm = plsc.sort_key_val(okey, iota)
    t1_v[...] = so
    prev = plsc.load_gather(t1_v, [jnp.maximum(iota - 1, 0)])
    fo = jnp.logical_or(iota == 0, so != prev)
    bse = plsc.cummax(jnp.where(fo, iota, 0))
    plsc.store_scatter(t2_v, [perm], iota - bse)
    rk = jnp.where(valid, t2_v[...], -1)
    nr = jnp.max(rk) + 1

    def round_body(r, _):
        msk = rk == jnp.full((16,), r, jnp.int32)

        def col_step(cs, _2):
            for j in range(16):
                jj = jnp.full((16,), cs * 16 + j, jnp.int32)
                hv = plsc.load_gather(rowbuf_v, [iota, jj], mask=msk)
                val = eg * hv if weighted else hv
                cur = plsc.load_gather(acc_v, [og, jj], mask=msk)
                plsc.store_scatter(acc_v, [og, jj], jnp.maximum(cur, val),
                                   mask=msk)
            return 0

        lax.fori_loop(0, D // 16, col_step, 0)
        return 0

    lax.fori_loop(0, nr, round_body, 0)


def _gat_scan_body(src_hbm, dst_hbm, asrc_hbm, adst_hbm, h_hbm, mv_hbm,
                   ninit_hbm, zrep0_hbm, maxout_hbm, z_hbm,
                   asrc_v, adst_v, acc_v, zrep_v, srcb_v, dstb_v,
                   ls_v, lo_v, le_v, idx_v, rowbuf_v, t1_v, t2_v, zout_v,
                   mvf_v, sem):
    cid = lax.axis_index("c")
    sid = lax.axis_index("s")
    base = sid * PT
    eoff = cid * E
    iota = lax.iota(jnp.int32, 16)
    pltpu.sync_copy(asrc_hbm.at[pl.ds(cid * NP, NP)], asrc_v)
    pltpu.sync_copy(adst_hbm.at[pl.ds(cid * NP, NP)], adst_v)
    pltpu.sync_copy(ninit_hbm, acc_v)
    pltpu.sync_copy(mv_hbm.at[pl.ds(cid * 16, 16)], mvf_v)
    pltpu.sync_copy(zrep0_hbm, zrep_v)
    mv = mvf_v[...]

    def drain_full(rp):
        _drain(ls_v, lo_v, le_v, idx_v, rowbuf_v, t1_v, t2_v, sem, h_hbm,
               acc_v, cid * N, rp, jnp.int32(16), True)
        return rp + 16

    def chunk(ci, carry):
        wpos, rpos = carry
        pltpu.sync_copy(src_hbm.at[pl.ds(eoff + ci * EC, EC)], srcb_v)
        pltpu.sync_copy(dst_hbm.at[pl.ds(eoff + ci * EC, EC)], dstb_v)
        for g in range(EC // 16):
            s = srcb_v[pl.ds(g * 16, 16)]
            d = dstb_v[pl.ds(g * 16, 16)]
            bb = jnp.full((16,), base, jnp.int32)
            own = jnp.logical_and(d >= bb, d < bb + PT)
            av = plsc.load_gather(asrc_v, [s])
            bv = plsc.load_gather(adst_v, [d])
            q = av + bv
            e = jnp.maximum(q, 0.0) + 0.2 * jnp.minimum(q, 0.0)
            expe = jnp.exp(e - mv)
            off = jnp.where(own, d - bb, 0)
            curz = plsc.load_gather(zrep_v, [iota, off], mask=own)
            plsc.store_scatter(zrep_v, [iota, off], curz + expe, mask=own)
            owni = own.astype(jnp.int32)
            pc = plsc.cumsum(owni)
            pos = (jnp.full((16,), wpos, jnp.int32) + pc - 1) & (RING - 1)
            plsc.store_scatter(ls_v, [pos], s, mask=own)
            plsc.store_scatter(lo_v, [pos], off, mask=own)
            plsc.store_scatter(le_v, [pos], expe, mask=own)
            wpos = wpos + jnp.max(pc)
            rpos = lax.cond(wpos - rpos >= 16, drain_full, lambda rp: rp, rpos)
        return wpos, rpos

    wpos, rpos = lax.fori_loop(0, E // EC, chunk,
                               (jnp.int32(0), jnp.int32(0)))

    def drain_tail(rp):
        _drain(ls_v, lo_v, le_v, idx_v, rowbuf_v, t1_v, t2_v, sem, h_hbm,
               acc_v, cid * N, rp, wpos - rp, True)
        return rp

    rpos = lax.cond(wpos > rpos, drain_tail, lambda rp: rp, rpos)

    def zfin(g, _):
        zacc = zrep_v[0, pl.ds(g * 16, 16)]
        for r in range(1, 16):
            zacc = zacc + zrep_v[r, pl.ds(g * 16, 16)]
        zout_v[pl.ds(g * 16, 16)] = zacc
        return 0

    lax.fori_loop(0, PT // 16, zfin, 0)
    pltpu.sync_copy(zout_v, z_hbm.at[pl.ds(cid * NP + base, PT)])
    pltpu.sync_copy(acc_v, maxout_hbm.at[pl.ds(cid * NP + base, PT)])


def _gat_sc(src2, dst2, asrc2, adst2, h2, mvec2):
    ninit = jnp.full((PT, D), NEG, jnp.float32)
    zrep0 = jnp.zeros((16, PT), jnp.float32)
    k = pl.kernel(
        _gat_scan_body,
        out_type=(jax.ShapeDtypeStruct((2 * NP, D), jnp.float32),
                  jax.ShapeDtypeStruct((2 * NP,), jnp.float32)),
        mesh=_mesh(),
        compiler_params=_params(),
        scratch_types=[
            pltpu.VMEM((NP,), jnp.float32),
            pltpu.VMEM((NP,), jnp.float32),
            pltpu.VMEM((PT, D), jnp.float32),
            pltpu.VMEM((16, PT), jnp.float32),
            pltpu.VMEM((EC,), jnp.int32),
            pltpu.VMEM((EC,), jnp.int32),
            pltpu.VMEM((RING,), jnp.int32),
            pltpu.VMEM((RING,), jnp.int32),
            pltpu.VMEM((RING,), jnp.float32),
            pltpu.VMEM((16,), jnp.int32),
            pltpu.VMEM((16, D), jnp.float32),
            pltpu.VMEM((16,), jnp.int32),
            pltpu.VMEM((16,), jnp.int32),
            pltpu.VMEM((PT,), jnp.float32),
            pltpu.VMEM((16,), jnp.float32),
            pltpu.SemaphoreType.DMA,
        ],
    )
    return k(src2, dst2, asrc2, adst2, h2, mvec2, ninit, zrep0)


def _segmax_scan_body(src_hbm, dst_hbm, x_hbm, ninit_hbm, out_hbm,
                      acc_v, srcb_v, dstb_v, ls_v, lo_v, idx_v, rowbuf_v,
                      t1_v, t2_v, sem):
    cid = lax.axis_index("c")
    sid = lax.axis_index("s")
    base = sid * PT
    eoff = cid * E
    pltpu.sync_copy(ninit_hbm, acc_v)

    def drain_full(rp):
        _drain(ls_v, lo_v, None, idx_v, rowbuf_v, t1_v, t2_v, sem, x_hbm,
               acc_v, cid * N, rp, jnp.int32(16), False)
        return rp + 16

    def chunk(ci, carry):
        wpos, rpos = carry
        pltpu.sync_copy(src_hbm.at[pl.ds(eoff + ci * EC, EC)], srcb_v)
        pltpu.sync_copy(dst_hbm.at[pl.ds(eoff + ci * EC, EC)], dstb_v)
        for g in range(EC // 16):
            s = srcb_v[pl.ds(g * 16, 16)]
            d = dstb_v[pl.ds(g * 16, 16)]
            bb = jnp.full((16,), base, jnp.int32)
            own = jnp.logical_and(d >= bb, d < bb + PT)
            off = jnp.where(own, d - bb, 0)
            owni = own.astype(jnp.int32)
            pc = plsc.cumsum(owni)
            pos = (jnp.full((16,), wpos, jnp.int32) + pc - 1) & (RING - 1)
            plsc.store_scatter(ls_v, [pos], s, mask=own)
            plsc.store_scatter(lo_v, [pos], off, mask=own)
            wpos = wpos + jnp.max(pc)
            rpos = lax.cond(wpos - rpos >= 16, drain_full, lambda rp: rp, rpos)
        return wpos, rpos

    wpos, rpos = lax.fori_loop(0, E // EC, chunk,
                               (jnp.int32(0), jnp.int32(0)))

    def drain_tail(rp):
        _drain(ls_v, lo_v, None, idx_v, rowbuf_v, t1_v, t2_v, sem, x_hbm,
               acc_v, cid * N, rp, wpos - rp, False)
        return rp

    rpos = lax.cond(wpos > rpos, drain_tail, lambda rp: rp, rpos)
    pltpu.sync_copy(acc_v, out_hbm.at[pl.ds(cid * NP + base, PT)])


def _segmax_sc(src2, dst2, x2):
    ninit = jnp.full((PT, D), NEG, jnp.float32)
    k = pl.kernel(
        _segmax_scan_body,
        out_type=jax.ShapeDtypeStruct((2 * NP, D), jnp.float32),
        mesh=_mesh(),
        compiler_params=_params(),
        scratch_types=[
            pltpu.VMEM((PT, D), jnp.float32),
            pltpu.VMEM((EC,), jnp.int32),
            pltpu.VMEM((EC,), jnp.int32),
            pltpu.VMEM((RING,), jnp.int32),
            pltpu.VMEM((RING,), jnp.int32),
            pltpu.VMEM((16,), jnp.int32),
            pltpu.VMEM((16, D), jnp.float32),
            pltpu.VMEM((16,), jnp.int32),
            pltpu.VMEM((16,), jnp.int32),
            pltpu.SemaphoreType.DMA,
        ],
    )
    return k(src2, dst2, x2, ninit)


def _segsum_body(src_hbm, dst_hbm, x_hbm, zrows_hbm, out_hbm,
                 srcb_v, dstb_v, rows_v, sem, shared_v):
    cid = lax.axis_index("c")
    sid = lax.axis_index("s")
    eoff = cid * E
    iota = lax.iota(jnp.int32, 16)
    sl = NP // 16
    pltpu.sync_copy(zrows_hbm, shared_v.at[pl.ds(sid * sl, sl)])
    plsc.subcore_barrier()
    nfull = (E // EC) // NT
    nch = jnp.where(sid < (E // EC) % NT, nfull + 1, nfull)

    def body(i, _):
        ci = sid + i * NT
        pltpu.sync_copy(src_hbm.at[pl.ds(eoff + ci * EC, EC)], srcb_v)
        pltpu.sync_copy(dst_hbm.at[pl.ds(eoff + ci * EC, EC)], dstb_v)
        for g in range(EC // 16):
            srcb_v[pl.ds(g * 16, 16)] = (srcb_v[pl.ds(g * 16, 16)] +
                                         jnp.full((16,), cid * N, jnp.int32))
        pltpu.async_copy(x_hbm.at[srcb_v], rows_v, sem).wait()
        pltpu.sync_copy(rows_v, shared_v.at[dstb_v], add=True)
        return 0

    lax.fori_loop(0, nch, body, 0)
    plsc.subcore_barrier()
    pltpu.sync_copy(shared_v.at[pl.ds(sid * sl, sl)],
                    out_hbm.at[pl.ds(cid * NP + sid * sl, sl)])


def _segsum_sc(src2, dst2, x2):
    zrows = jnp.zeros((NP // 16, D), jnp.float32)
    k = pl.kernel(
        _segsum_body,
        out_type=jax.ShapeDtypeStruct((2 * NP, D), jnp.float32),
        mesh=_mesh(),
        compiler_params=_params(),
        scratch_types=[
            pltpu.VMEM((EC,), jnp.int32),
            pltpu.VMEM((EC,), jnp.int32),
            pltpu.VMEM((EC, D), jnp.float32),
            pltpu.SemaphoreType.DMA,
            pltpu.VMEM_SHARED((NP, D), jnp.float32),
        ],
    )
    return k(src2, dst2, x2, zrows)


def _bn(x, g, b, eps=1e-5):
    mu = x.mean(axis=0)
    var = x.var(axis=0)
    return (x - mu) / jnp.sqrt(var + eps) * g + b


def _prelu(x, a):
    return jnp.maximum(x, 0.0) + a * jnp.minimum(x, 0.0)


def _lrelu(x):
    return jnp.maximum(x, 0.0) + 0.2 * jnp.minimum(x, 0.0)


def _pad_nodes(v):
    return jnp.zeros((NP,), jnp.float32).at[:N].set(v)


def _head_body(ae_ref, ge_ref, addf_ref, fc1w_ref, fc1b_ref, pr3_ref,
               fc2w_ref, fc2b_ref, out_ref):
    pool = jnp.concatenate([ae_ref[...], ge_ref[...], addf_ref[...]])[None, :]
    hid = pool @ fc1w_ref[...] + fc1b_ref[...][None, :]
    a3 = pr3_ref[0]
    hid = jnp.maximum(hid, 0.0) + a3 * jnp.minimum(hid, 0.0)
    out = jnp.exp(hid @ fc2w_ref[...] + fc2b_ref[...][None, :])
    out_ref[...] = out[0]


def kernel(A_x, A_edge_index, A_batch, A_W_gat, A_att_src, A_att_dst, A_b_gat, A_bn1_g, A_bn1_b, A_prelu1, A_W_sage_l, A_W_sage_r, A_b_sage, A_bn2_g, A_bn2_b, A_prelu2, A_Wp_rel, A_bp_rel, A_Wp_root, G_x, G_edge_index, G_batch, G_W_gat, G_att_src, G_att_dst, G_b_gat, G_bn1_g, G_bn1_b, G_prelu1, G_W_sage_l, G_W_sage_r, G_b_sage, G_bn2_g, G_bn2_b, G_prelu2, G_Wp_rel, G_bp_rel, G_Wp_root, add_features, fc1_W, fc1_b, prelu3, fc2_W, fc2_b):
    kw = dict(locals())
    pA = {k: v for k, v in kw.items() if k.startswith('A_')}
    pG = {k: v for k, v in kw.items() if k.startswith('G_')}
    towers = []
    src2 = jnp.concatenate([A_edge_index[0], G_edge_index[0]])
    dst2 = jnp.concatenate([A_edge_index[1], G_edge_index[1]])

    hs, asrcs, adsts, ms = [], [], [], []
    for p, pr in ((pA, 'A_'), (pG, 'G_')):
        x = p[pr + 'x']
        h = x @ p[pr + 'W_gat']
        a_src = (h * p[pr + 'att_src']).sum(axis=-1)
        a_dst = (h * p[pr + 'att_dst']).sum(axis=-1)
        hs.append(h)
        asrcs.append(a_src)
        adsts.append(a_dst)
        ms.append(_lrelu(a_src.max() + a_dst.max()))
    h2cat = jnp.concatenate(hs, axis=0)
    asrc2 = jnp.concatenate([_pad_nodes(a) for a in asrcs])
    adst2 = jnp.concatenate([_pad_nodes(a) for a in adsts])
    mvec2 = jnp.concatenate([jnp.full((16,), m, jnp.float32) for m in ms])

    maxout2, z2 = _gat_sc(src2, dst2, asrc2, adst2, h2cat, mvec2)

    outs = []
    for t, (p, pr) in enumerate(((pA, 'A_'), (pG, 'G_'))):
        maxout = maxout2[t * NP:t * NP + N]
        z = z2[t * NP:t * NP + N]
        expe_self = jnp.exp(_lrelu(asrcs[t] + adsts[t]) - ms[t])
        denom = z + expe_self + 1e-16 * jnp.exp(-ms[t])
        out = jnp.maximum(maxout, expe_self[:, None] * hs[t]) / denom[:, None]
        out = out + p[pr + 'b_gat']
        out = _prelu(_bn(out, p[pr + 'bn1_g'], p[pr + 'bn1_b']),
                     p[pr + 'prelu1'])
        outs.append(out)

    nbr2 = _segmax_sc(src2, dst2, jnp.concatenate(outs, axis=0))

    h2s = []
    for t, (p, pr) in enumerate(((pA, 'A_'), (pG, 'G_'))):
        nbr = nbr2[t * NP:t * NP + N]
        nbr = jnp.where(nbr < -1e37, 0.0, nbr)
        out = outs[t]
        h2 = (nbr @ p[pr + 'W_sage_l'] + p[pr + 'b_sage'] +
              out @ p[pr + 'W_sage_r'])
        h2 = _prelu(_bn(h2, p[pr + 'bn2_g'], p[pr + 'bn2_b']),
                    p[pr + 'prelu2'])
        h2s.append(h2)

    agg2 = _segsum_sc(src2, dst2, jnp.concatenate(h2s, axis=0))

    for t, (p, pr) in enumerate(((pA, 'A_'), (pG, 'G_'))):
        agg = agg2[t * NP:t * NP + N]
        h2 = h2s[t]
        score = (agg @ p[pr + 'Wp_rel'] + p[pr + 'bp_rel'] +
                 h2 @ p[pr + 'Wp_root']).reshape(-1)
        k = (N + 1) // 2
        _, perm = jax.lax.top_k(score, k)
        towers.append(jnp.max(h2[perm] * jnp.tanh(score[perm])[:, None],
                              axis=0))

    out = pl.pallas_call(
        _head_body,
        out_shape=jax.ShapeDtypeStruct((1,), jnp.float32),
    )(towers[0], towers[1], add_features, fc1_W, fc1_b, prelu3, fc2_W, fc2_b)
    return out


# speedup vs baseline: 1.7272x; 1.2923x over previous
import jax
import jax.numpy as jnp
from jax import lax
from jax.experimental import pallas as pl
from jax.experimental.pallas import tpu as pltpu
from jax.experimental.pallas import tpu_sc as plsc

N = 10000
E = 320000
D = 128
NT = 16
NP = 10240
PT = NP // NT
EC = 128
RING = 32
NEG = -3.4e38


def _mesh():
    return plsc.VectorSubcoreMesh(core_axis_name="c", subcore_axis_name="s")


def _params():
    return pltpu.CompilerParams(needs_layout_passes=False)


def _drain(ls_v, lo_v, le_v, idx_v, rowbuf_v, t1_v, t2_v, sem, x_hbm, acc_v,
           rowoff, rpos, nvalid, weighted):
    iota = lax.iota(jnp.int32, 16)
    ridx = (jnp.full((16,), rpos, jnp.int32) + iota) & (RING - 1)
    valid = iota < jnp.full((16,), nvalid, jnp.int32)
    sg = plsc.load_gather(ls_v, [ridx])
    og = plsc.load_gather(lo_v, [ridx])
    sg = jnp.where(valid, sg, 0)
    if weighted:
        eg = plsc.load_gather(le_v, [ridx])
    idx_v[...] = sg + jnp.full((16,), rowoff, jnp.int32)
    pltpu.async_copy(x_hbm.at[idx_v], rowbuf_v, sem).wait()
    okey = jnp.where(valid, og, PT + iota)
    so, perm = plsc.sort_key_val(okey, iota)
    t1_v[...] = so
    prev = plsc.load_gather(t1_v, [jnp.maximum(iota - 1, 0)])
    fo = jnp.logical_or(iota == 0, so != prev)
    bse = plsc.cummax(jnp.where(fo, iota, 0))
    plsc.store_scatter(t2_v, [perm], iota - bse)
    rk = jnp.where(valid, t2_v[...], -1)
    nr = jnp.max(rk) + 1

    def round_body(r, _):
        msk = rk == jnp.full((16,), r, jnp.int32)

        def col_step(cs, _2):
            for j in range(16):
                jj = jnp.full((16,), cs * 16 + j, jnp.int32)
                hv = plsc.load_gather(rowbuf_v, [iota, jj], mask=msk)
                val = eg * hv if weighted else hv
                cur = plsc.load_gather(acc_v, [og, jj], mask=msk)
                plsc.store_scatter(acc_v, [og, jj], jnp.maximum(cur, val),
                                   mask=msk)
            return 0

        lax.fori_loop(0, D // 16, col_step, 0)
        return 0

    lax.fori_loop(0, nr, round_body, 0)


def _gat_scan_body(src_hbm, dst_hbm, asrc_hbm, adst_hbm, h_hbm, mv_hbm,
                   ninit_hbm, zrep0_hbm, maxout_hbm, z_hbm,
                   asrc_v, adst_v, acc_v, zrep_v, srcb_v, dstb_v,
                   ls_v, lo_v, le_v, idx_v, rowbuf_v, t1_v, t2_v, zout_v,
                   mvf_v, sem, semc0, semc1):
    cid = lax.axis_index("c")
    sid = lax.axis_index("s")
    base = sid * PT
    eoff = cid * E
    iota = lax.iota(jnp.int32, 16)
    pltpu.sync_copy(asrc_hbm.at[pl.ds(cid * NP, NP)], asrc_v)
    pltpu.sync_copy(adst_hbm.at[pl.ds(cid * NP, NP)], adst_v)
    pltpu.sync_copy(ninit_hbm, acc_v)
    pltpu.sync_copy(mv_hbm.at[pl.ds(cid * 16, 16)], mvf_v)
    pltpu.sync_copy(zrep0_hbm, zrep_v)
    mv = mvf_v[...]

    def drain_full(rp):
        _drain(ls_v, lo_v, le_v, idx_v, rowbuf_v, t1_v, t2_v, sem, h_hbm,
               acc_v, cid * N, rp, jnp.int32(16), True)
        return rp + 16

    def start_chunk(ci, sref, dref, semc):
        pltpu.async_copy(src_hbm.at[pl.ds(eoff + ci * EC, EC)], sref, semc)
        pltpu.async_copy(dst_hbm.at[pl.ds(eoff + ci * EC, EC)], dref, semc)

    def wait_chunk(sref, dref, semc):
        pltpu.make_async_copy(src_hbm.at[pl.ds(0, EC)], sref, semc).wait()
        pltpu.make_async_copy(src_hbm.at[pl.ds(0, EC)], dref, semc).wait()

    start_chunk(0, srcb_v.at[0], dstb_v.at[0], semc0)
    start_chunk(1, srcb_v.at[1], dstb_v.at[1], semc1)

    def chunk(i, carry):
        wpos, rpos = carry
        for par, semc in ((0, semc0), (1, semc1)):
            ci = i * 2 + par
            wait_chunk(srcb_v.at[par], dstb_v.at[par], semc)

            def group(g, c2, par=par):
                wpos, rpos = c2
                s = srcb_v[par, pl.ds(g * 16, 16)]
                d = dstb_v[par, pl.ds(g * 16, 16)]
                bb = jnp.full((16,), base, jnp.int32)
                own = jnp.logical_and(d >= bb, d < bb + PT)
                av = plsc.load_gather(asrc_v, [s])
                bv = plsc.load_gather(adst_v, [d])
                q = av + bv
                e = jnp.maximum(q, 0.0) + 0.2 * jnp.minimum(q, 0.0)
                expe = jnp.exp(e - mv)
                off = jnp.where(own, d - bb, 0)
                curz = plsc.load_gather(zrep_v, [iota, off], mask=own)
                plsc.store_scatter(zrep_v, [iota, off], curz + expe, mask=own)
                owni = own.astype(jnp.int32)
                pc = plsc.cumsum(owni)
                pos = (jnp.full((16,), wpos, jnp.int32) + pc - 1) & (RING - 1)
                plsc.store_scatter(ls_v, [pos], s, mask=own)
                plsc.store_scatter(lo_v, [pos], off, mask=own)
                plsc.store_scatter(le_v, [pos], expe, mask=own)
                wpos = wpos + jnp.max(pc)
                rpos = lax.cond(wpos - rpos >= 16, drain_full,
                                lambda rp: rp, rpos)
                return wpos, rpos

            wpos, rpos = lax.fori_loop(0, EC // 16, group, (wpos, rpos))
            nxt = jnp.minimum(ci + 2, E // EC - 1)
            start_chunk(nxt, srcb_v.at[par], dstb_v.at[par], semc)
        return wpos, rpos

    wpos, rpos = lax.fori_loop(0, E // EC // 2, chunk,
                               (jnp.int32(0), jnp.int32(0)))
    wait_chunk(srcb_v.at[0], dstb_v.at[0], semc0)
    wait_chunk(srcb_v.at[1], dstb_v.at[1], semc1)

    def drain_tail(rp):
        _drain(ls_v, lo_v, le_v, idx_v, rowbuf_v, t1_v, t2_v, sem, h_hbm,
               acc_v, cid * N, rp, wpos - rp, True)
        return rp

    rpos = lax.cond(wpos > rpos, drain_tail, lambda rp: rp, rpos)

    def zfin(g, _):
        zacc = zrep_v[0, pl.ds(g * 16, 16)]
        for r in range(1, 8):
            zacc = (zacc + zrep_v[2 * r, pl.ds(g * 16, 16)] +
                    zrep_v[2 * r + 1, pl.ds(g * 16, 16)])
        zacc = zacc + zrep_v[1, pl.ds(g * 16, 16)]
        zout_v[pl.ds(g * 16, 16)] = zacc
        return 0

    lax.fori_loop(0, PT // 16, zfin, 0)
    pltpu.sync_copy(zout_v, z_hbm.at[pl.ds(cid * NP + base, PT)])
    pltpu.sync_copy(acc_v, maxout_hbm.at[pl.ds(cid * NP + base, PT)])


def _gat_sc(src2, dst2, asrc2, adst2, h2, mvec2):
    ninit = jnp.full((PT, D), NEG, jnp.float32)
    zrep0 = jnp.zeros((16, PT), jnp.float32)
    k = pl.kernel(
        _gat_scan_body,
        out_type=(jax.ShapeDtypeStruct((2 * NP, D), jnp.float32),
                  jax.ShapeDtypeStruct((2 * NP,), jnp.float32)),
        mesh=_mesh(),
        compiler_params=_params(),
        scratch_types=[
            pltpu.VMEM((NP,), jnp.float32),
            pltpu.VMEM((NP,), jnp.float32),
            pltpu.VMEM((PT, D), jnp.float32),
            pltpu.VMEM((16, PT), jnp.float32),
            pltpu.VMEM((2, EC), jnp.int32),
            pltpu.VMEM((2, EC), jnp.int32),
            pltpu.VMEM((RING,), jnp.int32),
            pltpu.VMEM((RING,), jnp.int32),
            pltpu.VMEM((RING,), jnp.float32),
            pltpu.VMEM((16,), jnp.int32),
            pltpu.VMEM((16, D), jnp.float32),
            pltpu.VMEM((16,), jnp.int32),
            pltpu.VMEM((16,), jnp.int32),
            pltpu.VMEM((PT,), jnp.float32),
            pltpu.VMEM((16,), jnp.float32),
            pltpu.SemaphoreType.DMA,
            pltpu.SemaphoreType.DMA,
            pltpu.SemaphoreType.DMA,
        ],
    )
    return k(src2, dst2, asrc2, adst2, h2, mvec2, ninit, zrep0)


def _segmax_scan_body(src_hbm, dst_hbm, x_hbm, ninit_hbm, out_hbm,
                      acc_v, srcb_v, dstb_v, ls_v, lo_v, idx_v, rowbuf_v,
                      t1_v, t2_v, sem, semc0, semc1):
    cid = lax.axis_index("c")
    sid = lax.axis_index("s")
    base = sid * PT
    eoff = cid * E
    pltpu.sync_copy(ninit_hbm, acc_v)

    def drain_full(rp):
        _drain(ls_v, lo_v, None, idx_v, rowbuf_v, t1_v, t2_v, sem, x_hbm,
               acc_v, cid * N, rp, jnp.int32(16), False)
        return rp + 16

    def start_chunk(ci, sref, dref, semc):
        pltpu.async_copy(src_hbm.at[pl.ds(eoff + ci * EC, EC)], sref, semc)
        pltpu.async_copy(dst_hbm.at[pl.ds(eoff + ci * EC, EC)], dref, semc)

    def wait_chunk(sref, dref, semc):
        pltpu.make_async_copy(src_hbm.at[pl.ds(0, EC)], sref, semc).wait()
        pltpu.make_async_copy(src_hbm.at[pl.ds(0, EC)], dref, semc).wait()

    start_chunk(0, srcb_v.at[0], dstb_v.at[0], semc0)
    start_chunk(1, srcb_v.at[1], dstb_v.at[1], semc1)

    def chunk(i, carry):
        wpos, rpos = carry
        for par, semc in ((0, semc0), (1, semc1)):
            ci = i * 2 + par
            wait_chunk(srcb_v.at[par], dstb_v.at[par], semc)
            for g in range(EC // 16):
                s = srcb_v[par, pl.ds(g * 16, 16)]
                d = dstb_v[par, pl.ds(g * 16, 16)]
                bb = jnp.full((16,), base, jnp.int32)
                own = jnp.logical_and(d >= bb, d < bb + PT)
                off = jnp.where(own, d - bb, 0)
                owni = own.astype(jnp.int32)
                pc = plsc.cumsum(owni)
                pos = (jnp.full((16,), wpos, jnp.int32) + pc - 1) & (RING - 1)
                plsc.store_scatter(ls_v, [pos], s, mask=own)
                plsc.store_scatter(lo_v, [pos], off, mask=own)
                wpos = wpos + jnp.max(pc)
                rpos = lax.cond(wpos - rpos >= 16, drain_full,
                                lambda rp: rp, rpos)
            nxt = jnp.minimum(ci + 2, E // EC - 1)
            start_chunk(nxt, srcb_v.at[par], dstb_v.at[par], semc)
        return wpos, rpos

    wpos, rpos = lax.fori_loop(0, E // EC // 2, chunk,
                               (jnp.int32(0), jnp.int32(0)))
    wait_chunk(srcb_v.at[0], dstb_v.at[0], semc0)
    wait_chunk(srcb_v.at[1], dstb_v.at[1], semc1)

    def drain_tail(rp):
        _drain(ls_v, lo_v, None, idx_v, rowbuf_v, t1_v, t2_v, sem, x_hbm,
               acc_v, cid * N, rp, wpos - rp, False)
        return rp

    rpos = lax.cond(wpos > rpos, drain_tail, lambda rp: rp, rpos)
    pltpu.sync_copy(acc_v, out_hbm.at[pl.ds(cid * NP + base, PT)])


def _segmax_sc(src2, dst2, x2):
    ninit = jnp.full((PT, D), NEG, jnp.float32)
    k = pl.kernel(
        _segmax_scan_body,
        out_type=jax.ShapeDtypeStruct((2 * NP, D), jnp.float32),
        mesh=_mesh(),
        compiler_params=_params(),
        scratch_types=[
            pltpu.VMEM((PT, D), jnp.float32),
            pltpu.VMEM((2, EC), jnp.int32),
            pltpu.VMEM((2, EC), jnp.int32),
            pltpu.VMEM((RING,), jnp.int32),
            pltpu.VMEM((RING,), jnp.int32),
            pltpu.VMEM((16,), jnp.int32),
            pltpu.VMEM((16, D), jnp.float32),
            pltpu.VMEM((16,), jnp.int32),
            pltpu.VMEM((16,), jnp.int32),
            pltpu.SemaphoreType.DMA,
            pltpu.SemaphoreType.DMA,
            pltpu.SemaphoreType.DMA,
        ],
    )
    return k(src2, dst2, x2, ninit)


def _segsum_body(src_hbm, dst_hbm, x_hbm, zrows_hbm, out_hbm,
                 srcb_v, dstb_v, rows_v, sem, shared_v):
    cid = lax.axis_index("c")
    sid = lax.axis_index("s")
    eoff = cid * E
    iota = lax.iota(jnp.int32, 16)
    sl = NP // 16
    pltpu.sync_copy(zrows_hbm, shared_v.at[pl.ds(sid * sl, sl)])
    plsc.subcore_barrier()
    nfull = (E // EC) // NT
    nch = jnp.where(sid < (E // EC) % NT, nfull + 1, nfull)

    def body(i, _):
        ci = sid + i * NT
        pltpu.sync_copy(src_hbm.at[pl.ds(eoff + ci * EC, EC)], srcb_v)
        pltpu.sync_copy(dst_hbm.at[pl.ds(eoff + ci * EC, EC)], dstb_v)
        for g in range(EC // 16):
            srcb_v[pl.ds(g * 16, 16)] = (srcb_v[pl.ds(g * 16, 16)] +
                                         jnp.full((16,), cid * N, jnp.int32))
        pltpu.async_copy(x_hbm.at[srcb_v], rows_v, sem).wait()
        pltpu.sync_copy(rows_v, shared_v.at[dstb_v], add=True)
        return 0

    lax.fori_loop(0, nch, body, 0)
    plsc.subcore_barrier()
    pltpu.sync_copy(shared_v.at[pl.ds(sid * sl, sl)],
                    out_hbm.at[pl.ds(cid * NP + sid * sl, sl)])


def _segsum_sc(src2, dst2, x2):
    zrows = jnp.zeros((NP // 16, D), jnp.float32)
    k = pl.kernel(
        _segsum_body,
        out_type=jax.ShapeDtypeStruct((2 * NP, D), jnp.float32),
        mesh=_mesh(),
        compiler_params=_params(),
        scratch_types=[
            pltpu.VMEM((EC,), jnp.int32),
            pltpu.VMEM((EC,), jnp.int32),
            pltpu.VMEM((EC, D), jnp.float32),
            pltpu.SemaphoreType.DMA,
            pltpu.VMEM_SHARED((NP, D), jnp.float32),
        ],
    )
    return k(src2, dst2, x2, zrows)


def _bn(x, g, b, eps=1e-5):
    mu = x.mean(axis=0)
    var = x.var(axis=0)
    return (x - mu) / jnp.sqrt(var + eps) * g + b


def _prelu(x, a):
    return jnp.maximum(x, 0.0) + a * jnp.minimum(x, 0.0)


def _lrelu(x):
    return jnp.maximum(x, 0.0) + 0.2 * jnp.minimum(x, 0.0)


def _pad_nodes(v):
    return jnp.zeros((NP,), jnp.float32).at[:N].set(v)


def _head_body(ae_ref, ge_ref, addf_ref, fc1w_ref, fc1b_ref, pr3_ref,
               fc2w_ref, fc2b_ref, out_ref):
    pool = jnp.concatenate([ae_ref[...], ge_ref[...], addf_ref[...]])[None, :]
    hid = pool @ fc1w_ref[...] + fc1b_ref[...][None, :]
    a3 = pr3_ref[0]
    hid = jnp.maximum(hid, 0.0) + a3 * jnp.minimum(hid, 0.0)
    out = jnp.exp(hid @ fc2w_ref[...] + fc2b_ref[...][None, :])
    out_ref[...] = out[0]


def kernel(A_x, A_edge_index, A_batch, A_W_gat, A_att_src, A_att_dst, A_b_gat, A_bn1_g, A_bn1_b, A_prelu1, A_W_sage_l, A_W_sage_r, A_b_sage, A_bn2_g, A_bn2_b, A_prelu2, A_Wp_rel, A_bp_rel, A_Wp_root, G_x, G_edge_index, G_batch, G_W_gat, G_att_src, G_att_dst, G_b_gat, G_bn1_g, G_bn1_b, G_prelu1, G_W_sage_l, G_W_sage_r, G_b_sage, G_bn2_g, G_bn2_b, G_prelu2, G_Wp_rel, G_bp_rel, G_Wp_root, add_features, fc1_W, fc1_b, prelu3, fc2_W, fc2_b):
    kw = dict(locals())
    pA = {k: v for k, v in kw.items() if k.startswith('A_')}
    pG = {k: v for k, v in kw.items() if k.startswith('G_')}
    towers = []
    src2 = jnp.concatenate([A_edge_index[0], G_edge_index[0]])
    dst2 = jnp.concatenate([A_edge_index[1], G_edge_index[1]])

    hs, asrcs, adsts, ms = [], [], [], []
    for p, pr in ((pA, 'A_'), (pG, 'G_')):
        x = p[pr + 'x']
        h = x @ p[pr + 'W_gat']
        a_src = (h * p[pr + 'att_src']).sum(axis=-1)
        a_dst = (h * p[pr + 'att_dst']).sum(axis=-1)
        hs.append(h)
        asrcs.append(a_src)
        adsts.append(a_dst)
        ms.append(_lrelu(a_src.max() + a_dst.max()))
    h2cat = jnp.concatenate(hs, axis=0)
    asrc2 = jnp.concatenate([_pad_nodes(a) for a in asrcs])
    adst2 = jnp.concatenate([_pad_nodes(a) for a in adsts])
    mvec2 = jnp.concatenate([jnp.full((16,), m, jnp.float32) for m in ms])

    maxout2, z2 = _gat_sc(src2, dst2, asrc2, adst2, h2cat, mvec2)

    outs = []
    for t, (p, pr) in enumerate(((pA, 'A_'), (pG, 'G_'))):
        maxout = maxout2[t * NP:t * NP + N]
        z = z2[t * NP:t * NP + N]
        expe_self = jnp.exp(_lrelu(asrcs[t] + adsts[t]) - ms[t])
        denom = z + expe_self + 1e-16 * jnp.exp(-ms[t])
        out = jnp.maximum(maxout, expe_self[:, None] * hs[t]) / denom[:, None]
        out = out + p[pr + 'b_gat']
        out = _prelu(_bn(out, p[pr + 'bn1_g'], p[pr + 'bn1_b']),
                     p[pr + 'prelu1'])
        outs.append(out)

    nbr2 = _segmax_sc(src2, dst2, jnp.concatenate(outs, axis=0))

    h2s = []
    for t, (p, pr) in enumerate(((pA, 'A_'), (pG, 'G_'))):
        nbr = nbr2[t * NP:t * NP + N]
        nbr = jnp.where(nbr < -1e37, 0.0, nbr)
        out = outs[t]
        h2 = (nbr @ p[pr + 'W_sage_l'] + p[pr + 'b_sage'] +
              out @ p[pr + 'W_sage_r'])
        h2 = _prelu(_bn(h2, p[pr + 'bn2_g'], p[pr + 'bn2_b']),
                    p[pr + 'prelu2'])
        h2s.append(h2)

    agg2 = _segsum_sc(src2, dst2, jnp.concatenate(h2s, axis=0))

    for t, (p, pr) in enumerate(((pA, 'A_'), (pG, 'G_'))):
        agg = agg2[t * NP:t * NP + N]
        h2 = h2s[t]
        score = (agg @ p[pr + 'Wp_rel'] + p[pr + 'bp_rel'] +
                 h2 @ p[pr + 'Wp_root']).reshape(-1)
        k = (N + 1) // 2
        _, perm = jax.lax.top_k(score, k)
        towers.append(jnp.max(h2[perm] * jnp.tanh(score[perm])[:, None],
                              axis=0))

    out = pl.pallas_call(
        _head_body,
        out_shape=jax.ShapeDtypeStruct((1,), jnp.float32),
    )(towers[0], towers[1], add_features, fc1_W, fc1_b, prelu3, fc2_W, fc2_b)
    return out
